# baseline (speedup 1.0000x reference)
import functools

import jax
import jax.numpy as jnp
from jax import lax
from jax.experimental import pallas as pl
from jax.experimental.pallas import tpu as pltpu
from jax.experimental.pallas import tpu_sc as plsc

ROWS = 128
COLS = 32768
L = 16
SLICES = COLS // L
UNROLL = 16
C_UNROLL = 8
NC = 2
NS = 16
NW = NC * NS
ROWS_PER = ROWS // NW

_NEG = float("-inf")


def _compute_tau(buf, cand):
    def max_body(i, acc):
        for j in range(UNROLL):
            acc = jnp.maximum(acc, buf[pl.ds(2 * (i * UNROLL + j) * L, L)])
        return acc
    acc0 = jnp.full((L,), _NEG, dtype=jnp.float32)
    colmax = lax.fori_loop(0, SLICES // UNROLL // 2, max_body, acc0)
    m = jnp.max(colmax)
    t0 = jnp.broadcast_to(m, (L,)) - 1.0

    iota = lax.iota(jnp.int32, L)
    i16 = jnp.full((L,), 16, dtype=jnp.int32)
    i0 = jnp.zeros((L,), dtype=jnp.int32)
    def c_body(s_idx, base16):
        v = buf[pl.ds(s_idx * L, L)]
        mask = v > t0
        cnt = plsc.all_reduce_population_count(mask)
        anyb = cnt > 0
        plsc.store_scatter(cand, [base16 + iota], v, mask=anyb)
        return base16 + jnp.where(anyb, i16, i0)
    base16 = plsc.parallel_loop(
        0, SLICES, unroll=C_UNROLL, carry=jnp.zeros((L,), jnp.int32)
    )(c_body)
    kc = jnp.max(base16)

    def sum_count(t):
        def body(i, carry):
            s, k = carry
            v = cand[pl.ds(i * L, L)]
            mask = v > t
            s = s + jnp.where(mask, v, 0.0)
            k = k + jnp.where(mask, 1.0, 0.0)
            return s, k
        z16 = jnp.zeros((L,), dtype=jnp.float32)
        s, k = lax.fori_loop(0, lax.div(kc, L), body, (z16, z16))
        return jnp.sum(s), jnp.sum(k)

    def cond(c):
        return jnp.logical_not(c[1])

    def step(c):
        t, _ = c
        s, k = sum_count(t)
        t_new = (jnp.broadcast_to(s, (L,)) - 1.0) / jnp.broadcast_to(k, (L,))
        t_up = jnp.maximum(t, t_new)
        return t_up, jnp.all(t_up == t)

    tau, _ = lax.while_loop(cond, step, (t0, False))
    return tau


def _sparsemax_body(logits_hbm, out_hbm, buf_a, buf_b, cand,
                    sem_a, sem_b, sem_o):
    wid = lax.axis_index("s") * NC + lax.axis_index("c")
    base_row = wid * ROWS_PER
    bufs = [buf_a, buf_b]
    sems = [sem_a, sem_b]

    def start_in(r):
        return pltpu.async_copy(logits_hbm.at[base_row + r], bufs[r % 2],
                                sems[r % 2])

    handles = [start_in(0), start_in(1)]
    out_h = None
    for r in range(ROWS_PER):
        buf = bufs[r % 2]
        handles[r % 2].wait()
        tau = _compute_tau(buf, cand)
        if out_h is not None:
            out_h.wait()
            if r + 1 < ROWS_PER:
                handles[(r + 1) % 2] = start_in(r + 1)

        def relu_body(i, c):
            for j in range(UNROLL):
                idx = pl.ds((i * UNROLL + j) * L, L)
                buf[idx] = jnp.maximum(buf[idx] - tau, 0.0)
            return c
        lax.fori_loop(0, SLICES // UNROLL, relu_body, 0)

        out_h = pltpu.async_copy(buf, out_hbm.at[base_row + r], sem_o)
    out_h.wait()


@jax.jit
def _sparsemax_sc(logits):
    mesh = plsc.VectorSubcoreMesh(core_axis_name="c", subcore_axis_name="s")
    kfn = functools.partial(
        pl.kernel,
        mesh=mesh,
        out_type=jax.ShapeDtypeStruct((ROWS, COLS), jnp.float32),
        scratch_types=[
            pltpu.VMEM((COLS,), jnp.float32),
            pltpu.VMEM((COLS,), jnp.float32),
            pltpu.VMEM((COLS,), jnp.float32),
            pltpu.SemaphoreType.DMA,
            pltpu.SemaphoreType.DMA,
            pltpu.SemaphoreType.DMA,
        ],
        compiler_params=pltpu.CompilerParams(needs_layout_passes=False),
    )(_sparsemax_body)
    return kfn(logits)


def kernel(logits):
    return _sparsemax_sc(logits.astype(jnp.float32))


# speedup vs baseline: 1.1815x; 1.1815x over previous
import functools

import jax
import jax.numpy as jnp
from jax import lax
from jax.experimental import pallas as pl
from jax.experimental.pallas import tpu as pltpu
from jax.experimental.pallas import tpu_sc as plsc

ROWS = 128
COLS = 32768
L = 16
SLICES = COLS // L
UNROLL = 16
C_UNROLL = 8
NC = 2
NS = 16
NW = NC * NS
ROWS_PER = ROWS // NW

_NEG = float("-inf")


def _compute_tau(buf, cand):
    def max_body(i, acc):
        for j in range(UNROLL):
            acc = jnp.maximum(acc, buf[pl.ds((i * UNROLL + j) * L, L)])
        return acc
    acc0 = jnp.full((L,), _NEG, dtype=jnp.float32)
    colmax = lax.fori_loop(0, SLICES // UNROLL, max_body, acc0)
    m = jnp.max(colmax)
    t0 = jnp.broadcast_to(m, (L,)) - 1.0

    iota = lax.iota(jnp.int32, L)
    i16 = jnp.full((L,), 16, dtype=jnp.int32)
    i0 = jnp.zeros((L,), dtype=jnp.int32)
    def c_body(s_idx, base16i):
        v = buf[pl.ds(s_idx * L, L)]
        mask = v > t0
        cnt = plsc.all_reduce_population_count(mask)
        anyb = cnt > 0
        plsc.store_scatter(cand, [base16i], v, mask=anyb)
        return base16i + jnp.where(anyb, i16, i0)
    base16i = plsc.parallel_loop(
        0, SLICES, unroll=C_UNROLL, carry=iota
    )(c_body)
    kc = jnp.max(base16i) - (L - 1)

    def sum_count(t):
        def body(i, carry):
            s, k = carry
            v = cand[pl.ds(i * L, L)]
            mask = v > t
            s = s + jnp.where(mask, v, 0.0)
            k = k + jnp.where(mask, 1.0, 0.0)
            return s, k
        z16 = jnp.zeros((L,), dtype=jnp.float32)
        s, k = lax.fori_loop(0, lax.div(kc, L), body, (z16, z16))
        return jnp.sum(s), jnp.sum(k)

    def cond(c):
        return jnp.logical_not(c[1])

    def step(c):
        t, _ = c
        s, k = sum_count(t)
        t_new = (jnp.broadcast_to(s, (L,)) - 1.0) / jnp.broadcast_to(k, (L,))
        t_up = jnp.maximum(t, t_new)
        return t_up, jnp.all(t_up == t)

    tau, _ = lax.while_loop(cond, step, (t0, False))
    return tau


def _sparsemax_body(logits_hbm, out_hbm, buf_a, buf_b, cand,
                    sem_a, sem_b, sem_o):
    wid = lax.axis_index("s") * NC + lax.axis_index("c")
    base_row = wid * ROWS_PER
    bufs = [buf_a, buf_b]
    sems = [sem_a, sem_b]

    def start_in(r):
        return pltpu.async_copy(logits_hbm.at[base_row + r], bufs[r % 2],
                                sems[r % 2])

    handles = [start_in(0), start_in(1)]
    out_h = None
    for r in range(ROWS_PER):
        buf = bufs[r % 2]
        handles[r % 2].wait()
        tau = _compute_tau(buf, cand)
        if out_h is not None:
            out_h.wait()
            if r + 1 < ROWS_PER:
                handles[(r + 1) % 2] = start_in(r + 1)

        def relu_body(i, c):
            for j in range(UNROLL):
                idx = pl.ds((i * UNROLL + j) * L, L)
                buf[idx] = jnp.maximum(buf[idx] - tau, 0.0)
            return c
        lax.fori_loop(0, SLICES // UNROLL, relu_body, 0)

        out_h = pltpu.async_copy(buf, out_hbm.at[base_row + r], sem_o)
    out_h.wait()


@jax.jit
def _sparsemax_sc(logits):
    mesh = plsc.VectorSubcoreMesh(core_axis_name="c", subcore_axis_name="s")
    kfn = functools.partial(
        pl.kernel,
        mesh=mesh,
        out_type=jax.ShapeDtypeStruct((ROWS, COLS), jnp.float32),
        scratch_types=[
            pltpu.VMEM((COLS,), jnp.float32),
            pltpu.VMEM((COLS,), jnp.float32),
            pltpu.VMEM((COLS,), jnp.float32),
            pltpu.SemaphoreType.DMA,
            pltpu.SemaphoreType.DMA,
            pltpu.SemaphoreType.DMA,
        ],
        compiler_params=pltpu.CompilerParams(needs_layout_passes=False),
    )(_sparsemax_body)
    return kfn(logits)


def kernel(logits):
    return _sparsemax_sc(logits.astype(jnp.float32))


# speedup vs baseline: 1.2020x; 1.0174x over previous
import functools

import jax
import jax.numpy as jnp
from jax import lax
from jax.experimental import pallas as pl
from jax.experimental.pallas import tpu as pltpu
from jax.experimental.pallas import tpu_sc as plsc

ROWS = 128
COLS = 32768
L = 16
SLICES = COLS // L
UNROLL = 16
C_UNROLL = 8
NC = 2
NS = 16
NW = NC * NS
ROWS_PER = ROWS // NW

_NEG = float("-inf")


def _compute_tau(buf, cand):
    def max_body(i, acc):
        for j in range(UNROLL):
            acc = jnp.maximum(acc, buf[pl.ds((i * UNROLL + j) * L, L)])
        return acc
    acc0 = jnp.full((L,), _NEG, dtype=jnp.float32)
    colmax = lax.fori_loop(0, SLICES // UNROLL, max_body, acc0)
    m = jnp.max(colmax)
    t0 = jnp.broadcast_to(m, (L,)) - 1.0

    iota = lax.iota(jnp.int32, L)
    i16 = jnp.full((L,), 16, dtype=jnp.int32)
    i0 = jnp.zeros((L,), dtype=jnp.int32)
    def c_body(s_idx, base16i):
        v = buf[pl.ds(s_idx * L, L)]
        mask = v > t0
        cnt = plsc.all_reduce_population_count(mask)
        anyb = cnt > 0
        plsc.store_scatter(cand, [base16i], v, mask=anyb)
        return base16i + jnp.where(anyb, i16, i0)
    base16i = plsc.parallel_loop(
        0, SLICES, unroll=C_UNROLL, carry=iota
    )(c_body)
    kc = jnp.max(base16i) - (L - 1)

    def sum_count(t):
        def body(i, carry):
            s, k = carry
            v = cand[pl.ds(i * L, L)]
            mask = v > t
            s = s + jnp.where(mask, v, 0.0)
            k = k + jnp.where(mask, 1.0, 0.0)
            return s, k
        z16 = jnp.zeros((L,), dtype=jnp.float32)
        s, k = lax.fori_loop(0, lax.div(kc, L), body, (z16, z16))
        return jnp.sum(s), jnp.sum(k)

    def cond(c):
        return jnp.logical_not(c[1])

    def step(c):
        t, _ = c
        s, k = sum_count(t)
        t_new = (jnp.broadcast_to(s, (L,)) - 1.0) / jnp.broadcast_to(k, (L,))
        t_up = jnp.maximum(t, t_new)
        return t_up, jnp.all(t_up == t)

    tau, _ = lax.while_loop(cond, step, (t0, False))
    return tau


def _sparsemax_body(logits_hbm, out_hbm, buf_a, buf_b, cand,
                    sem_a, sem_b, sem_o):
    wid = lax.axis_index("s") * NC + lax.axis_index("c")
    base_row = wid * ROWS_PER
    bufs = [buf_a, buf_b]
    sems = [sem_a, sem_b]

    def start_in(r):
        return pltpu.async_copy(logits_hbm.at[base_row + r], bufs[r % 2],
                                sems[r % 2])

    handles = [start_in(0), None]
    out_h = None
    for r in range(ROWS_PER):
        buf = bufs[r % 2]
        handles[r % 2].wait()
        if r == 0:
            handles[1] = start_in(1)
        tau = _compute_tau(buf, cand)
        if out_h is not None:
            out_h.wait()
            if r + 1 < ROWS_PER:
                handles[(r + 1) % 2] = start_in(r + 1)

        def relu_body(i, c):
            for j in range(UNROLL):
                idx = pl.ds((i * UNROLL + j) * L, L)
                buf[idx] = jnp.maximum(buf[idx] - tau, 0.0)
            return c
        lax.fori_loop(0, SLICES // UNROLL, relu_body, 0)

        out_h = pltpu.async_copy(buf, out_hbm.at[base_row + r], sem_o)
    out_h.wait()


@jax.jit
def _sparsemax_sc(logits):
    mesh = plsc.VectorSubcoreMesh(core_axis_name="c", subcore_axis_name="s")
    kfn = functools.partial(
        pl.kernel,
        mesh=mesh,
        out_type=jax.ShapeDtypeStruct((ROWS, COLS), jnp.float32),
        scratch_types=[
            pltpu.VMEM((COLS,), jnp.float32),
            pltpu.VMEM((COLS,), jnp.float32),
            pltpu.VMEM((COLS,), jnp.float32),
            pltpu.SemaphoreType.DMA,
            pltpu.SemaphoreType.DMA,
            pltpu.SemaphoreType.DMA,
        ],
        compiler_params=pltpu.CompilerParams(needs_layout_passes=False),
    )(_sparsemax_body)
    return kfn(logits)


def kernel(logits):
    return _sparsemax_sc(logits.astype(jnp.float32))
